# baseline (speedup 1.0000x reference)
import jax
import jax.numpy as jnp
from jax import lax
from jax.experimental import pallas as pl
from jax.experimental.pallas import tpu as pltpu
from jax.experimental.pallas import tpu_sc as plsc

NUM_CARDS = 100000
CARD_ID_DIM = 64
HIDDEN_DIM = 128
BATCH = 4096
SEQ_LEN = 200
FEAT_EXTRA = 11
TOTAL = BATCH * SEQ_LEN

NC = 2
NS = 16
NW = NC * NS
CHUNK = 128

K = 4
NB = BATCH // 512
NB_K = NB // K
ROWS_K = TOTAL // K
PER_W = ROWS_K // NW
NCHUNK = PER_W // CHUNK

BB = 512
SB = 8
HALF = BB // 2


def _gather_body(ids_hbm, table_hbm, out_hbm, idx_v, rows_v, sem_i, sem_g):
    wid = lax.axis_index("s") * NC + lax.axis_index("c")
    base = wid * PER_W

    pltpu.async_copy(ids_hbm.at[wid], idx_v, sem_i).wait()

    def body(i, _):
        pltpu.async_copy(table_hbm.at[idx_v.at[i]], rows_v, sem_g).wait()
        pltpu.sync_copy(rows_v, out_hbm.at[pl.ds(base + i * CHUNK, CHUNK)])
        return ()

    lax.fori_loop(0, NCHUNK, body, (), unroll=False)


def _sc_gather(ids3, table):
    mesh = plsc.VectorSubcoreMesh(
        core_axis_name="c", subcore_axis_name="s", num_cores=NC, num_subcores=NS
    )
    return pl.kernel(
        _gather_body,
        out_type=jax.ShapeDtypeStruct((ROWS_K, CARD_ID_DIM), jnp.float32),
        mesh=mesh,
        compiler_params=pltpu.CompilerParams(use_tc_tiling_on_sc=False),
        scratch_types=[
            pltpu.VMEM((NCHUNK, CHUNK), jnp.int32),
            pltpu.VMEM((CHUNK, CARD_ID_DIM), jnp.float32),
            pltpu.SemaphoreType.DMA,
            pltpu.SemaphoreType.DMA,
        ],
    )(ids3, table)


def _mm_body(g_ref, f_ref, wid_ref, wf_ref, b_ref, o_ref):
    bvec = b_ref[...]
    wid = wid_ref[...]
    wf = wf_ref[...]
    for s in range(SB):
        g = g_ref[s]
        oe = jnp.dot(g[:, :CARD_ID_DIM], wid, preferred_element_type=jnp.float32)
        oo = jnp.dot(g[:, CARD_ID_DIM:], wid, preferred_element_type=jnp.float32)
        fts = f_ref[:, s, :]
        of = lax.dot_general(
            fts, wf,
            dimension_numbers=(((0,), (0,)), ((), ())),
            preferred_element_type=jnp.float32,
        )
        o_ref[0:HALF, s, :] = oe + of[0:HALF] + bvec
        o_ref[HALF:BB, s, :] = oo + of[HALF:BB] + bvec


def _mm_body_acc(g_ref, f_ref, wid_ref, wf_ref, b_ref, _prev_ref, o_ref):
    _mm_body(g_ref, f_ref, wid_ref, wf_ref, b_ref, o_ref)


def _tc_project_chunk(k, g3, ft, wid_t, wf_t, b2, prev):
    grid = (NB_K, SEQ_LEN // SB)
    in_specs = [
        pl.BlockSpec((SB, HALF, HIDDEN_DIM), lambda ib, isq: (isq, ib, 0)),
        pl.BlockSpec(
            (FEAT_EXTRA, SB, BB), lambda ib, isq, k=k: (0, isq, k * NB_K + ib)
        ),
        pl.BlockSpec((CARD_ID_DIM, HIDDEN_DIM), lambda ib, isq: (0, 0)),
        pl.BlockSpec((FEAT_EXTRA, HIDDEN_DIM), lambda ib, isq: (0, 0)),
        pl.BlockSpec((1, HIDDEN_DIM), lambda ib, isq: (0, 0)),
    ]
    out_spec = pl.BlockSpec(
        (BB, SB, HIDDEN_DIM), lambda ib, isq, k=k: (k * NB_K + ib, isq, 0)
    )
    out_shape = jax.ShapeDtypeStruct((BATCH, SEQ_LEN, HIDDEN_DIM), jnp.float32)
    if prev is None:
        return pl.pallas_call(
            _mm_body,
            grid=grid,
            in_specs=in_specs,
            out_specs=out_spec,
            out_shape=out_shape,
        )(g3, ft, wid_t, wf_t, b2)
    return pl.pallas_call(
        _mm_body_acc,
        grid=grid,
        in_specs=in_specs + [pl.BlockSpec(memory_space=pl.ANY)],
        out_specs=out_spec,
        out_shape=out_shape,
        input_output_aliases={5: 0},
    )(g3, ft, wid_t, wf_t, b2, prev)


@jax.jit
def _run(ids_perm, ft, table, wid_t, wf_t, b2):
    gs = [_sc_gather(ids_perm[k], table) for k in range(K)]
    out = None
    for k in range(K):
        g3 = gs[k].reshape(SEQ_LEN, BATCH // (2 * K), HIDDEN_DIM)
        out = _tc_project_chunk(k, g3, ft, wid_t, wf_t, b2, out)
    return out


def kernel(card_ids, card_features, table, W, b):
    idsT = card_ids.T.astype(jnp.int32)
    perm = (
        idsT.reshape(SEQ_LEN, NB, 2, HALF)
        .transpose(1, 0, 3, 2)
        .reshape(K, NB_K, SEQ_LEN, HALF, 2)
        .transpose(0, 2, 1, 3, 4)
        .reshape(K, NW, NCHUNK, CHUNK)
    )
    ft = jnp.transpose(card_features, (2, 1, 0))
    wid_t = W[:, :CARD_ID_DIM].T
    wf_t = W[:, CARD_ID_DIM:].T
    b2 = b.reshape(1, HIDDEN_DIM)
    return _run(perm, ft, table, wid_t, wf_t, b2)


# speedup vs baseline: 1.7930x; 1.7930x over previous
import functools

import jax
import jax.numpy as jnp
from jax import lax
from jax.experimental import pallas as pl
from jax.experimental.pallas import tpu as pltpu
from jax.experimental.pallas import tpu_sc as plsc

NUM_CARDS = 100000
CARD_ID_DIM = 64
HIDDEN_DIM = 128
BATCH = 4096
SEQ_LEN = 200
FEAT_EXTRA = 11
TOTAL = BATCH * SEQ_LEN

NC = 2
NS = 16
NW = NC * NS
CHUNK = 128

K = 4
NB = BATCH // 512
NB_K = NB // K
BPK = BATCH // K
ROWS_K = TOTAL // K
PER_W = ROWS_K // NW
NCHUNK = PER_W // CHUNK
SROWS = 8

BB = 512
SB = 8
HALF = BB // 2


def _make_gather_body(kofs):
    def _gather_body(ids_hbm, table_hbm, out_hbm, sids, idx_v, rows_v, sem_g):
        _gather_impl(kofs, ids_hbm, table_hbm, out_hbm, sids, idx_v, rows_v, sem_g)
    return _gather_body


def _gather_impl(kofs, ids_hbm, table_hbm, out_hbm, sids, idx_v, rows_v, sem_g):
    wid = lax.axis_index("s") * NC + lax.axis_index("c")
    base = wid * PER_W
    s_lo = jnp.minimum(base >> 10, SEQ_LEN - SROWS)

    pltpu.sync_copy(ids_hbm.at[pl.ds(s_lo, SROWS), pl.ds(kofs, BPK)], sids)

    lanes = lax.iota(jnp.int32, 16)
    idx_lo = lanes >> 1
    idx_hi = idx_lo + 8
    even = (lanes & 1) == 0
    dn = lax.GatherDimensionNumbers(
        offset_dims=(), collapsed_slice_dims=(0,), start_index_map=(0,)
    )

    def rgather(vec, idx):
        return lax.gather(
            vec, idx[:, None], dn, (1,),
            mode=lax.GatherScatterMode.PROMISE_IN_BOUNDS,
        )

    def build(i, _):
        flat = base + i * CHUNK
        srel = (flat >> 10) - s_lo
        rem_base = flat & 1023
        ibl = rem_base >> 9
        j0 = (rem_base >> 1) & 255
        col_a = ibl * 512 + j0
        col_b = col_a + 256
        for p in range(4):
            a16 = sids[srel, pl.ds(col_a + p * 16, 16)]
            b16 = sids[srel, pl.ds(col_b + p * 16, 16)]
            lo = jnp.where(even, rgather(a16, idx_lo), rgather(b16, idx_lo))
            hi = jnp.where(even, rgather(a16, idx_hi), rgather(b16, idx_hi))
            idx_v[i, pl.ds(p * 32, 16)] = lo
            idx_v[i, pl.ds(p * 32 + 16, 16)] = hi
        return ()

    lax.fori_loop(0, NCHUNK, build, (), unroll=False)

    def body(i, _):
        pltpu.async_copy(table_hbm.at[idx_v.at[i]], rows_v, sem_g).wait()
        pltpu.sync_copy(rows_v, out_hbm.at[pl.ds(base + i * CHUNK, CHUNK)])
        return ()

    lax.fori_loop(0, NCHUNK, body, (), unroll=False)


def _sc_gather(k, ids2, table):
    mesh = plsc.VectorSubcoreMesh(
        core_axis_name="c", subcore_axis_name="s", num_cores=NC, num_subcores=NS
    )
    return pl.kernel(
        _make_gather_body(k * BPK),
        out_type=jax.ShapeDtypeStruct((ROWS_K, CARD_ID_DIM), jnp.float32),
        mesh=mesh,
        compiler_params=pltpu.CompilerParams(use_tc_tiling_on_sc=False),
        scratch_types=[
            pltpu.VMEM((SROWS, BPK), jnp.int32),
            pltpu.VMEM((NCHUNK, CHUNK), jnp.int32),
            pltpu.VMEM((CHUNK, CARD_ID_DIM), jnp.float32),
            pltpu.SemaphoreType.DMA,
        ],
    )(ids2, table)


def _mm_body(g_ref, f_ref, wid_ref, wf_ref, b_ref, o_ref):
    bvec = b_ref[...]
    wid = wid_ref[...]
    wf = wf_ref[...]
    for s in range(SB):
        g = g_ref[s]
        oe = jnp.dot(g[:, :CARD_ID_DIM], wid, preferred_element_type=jnp.float32)
        oo = jnp.dot(g[:, CARD_ID_DIM:], wid, preferred_element_type=jnp.float32)
        fts = f_ref[:, s, :]
        of = lax.dot_general(
            fts, wf,
            dimension_numbers=(((0,), (0,)), ((), ())),
            preferred_element_type=jnp.float32,
        )
        o_ref[0:HALF, s, :] = oe + of[0:HALF] + bvec
        o_ref[HALF:BB, s, :] = oo + of[HALF:BB] + bvec


def _mm_body_acc(g_ref, f_ref, wid_ref, wf_ref, b_ref, _prev_ref, o_ref):
    _mm_body(g_ref, f_ref, wid_ref, wf_ref, b_ref, o_ref)


def _tc_project_chunk(k, g3, ft, wid_t, wf_t, b2, prev):
    grid = (NB_K, SEQ_LEN // SB)
    in_specs = [
        pl.BlockSpec((SB, HALF, HIDDEN_DIM), lambda ib, isq: (isq, ib, 0)),
        pl.BlockSpec(
            (FEAT_EXTRA, SB, BB), lambda ib, isq, k=k: (0, isq, k * NB_K + ib)
        ),
        pl.BlockSpec((CARD_ID_DIM, HIDDEN_DIM), lambda ib, isq: (0, 0)),
        pl.BlockSpec((FEAT_EXTRA, HIDDEN_DIM), lambda ib, isq: (0, 0)),
        pl.BlockSpec((1, HIDDEN_DIM), lambda ib, isq: (0, 0)),
    ]
    out_spec = pl.BlockSpec(
        (BB, SB, HIDDEN_DIM), lambda ib, isq, k=k: (k * NB_K + ib, isq, 0)
    )
    out_shape = jax.ShapeDtypeStruct((BATCH, SEQ_LEN, HIDDEN_DIM), jnp.float32)
    if prev is None:
        return pl.pallas_call(
            _mm_body,
            grid=grid,
            in_specs=in_specs,
            out_specs=out_spec,
            out_shape=out_shape,
        )(g3, ft, wid_t, wf_t, b2)
    return pl.pallas_call(
        _mm_body_acc,
        grid=grid,
        in_specs=in_specs + [pl.BlockSpec(memory_space=pl.ANY)],
        out_specs=out_spec,
        out_shape=out_shape,
        input_output_aliases={5: 0},
    )(g3, ft, wid_t, wf_t, b2, prev)


@jax.jit
def _run(ids2, ft, table, wid_t, wf_t, b2):
    gs = [_sc_gather(k, ids2, table) for k in range(K)]
    out = None
    for k in range(K):
        g3 = gs[k].reshape(SEQ_LEN, BATCH // (2 * K), HIDDEN_DIM)
        out = _tc_project_chunk(k, g3, ft, wid_t, wf_t, b2, out)
    return out


def kernel(card_ids, card_features, table, W, b):
    ids2 = card_ids.T.astype(jnp.int32)
    ft = jnp.transpose(card_features, (2, 1, 0))
    wid_t = W[:, :CARD_ID_DIM].T
    wf_t = W[:, CARD_ID_DIM:].T
    b2 = b.reshape(1, HIDDEN_DIM)
    return _run(ids2, ft, table, wid_t, wf_t, b2)
